# baseline (speedup 1.0000x reference)
import functools

import jax
import jax.numpy as jnp
from jax import lax
from jax.experimental import pallas as pl
from jax.experimental.pallas import tpu as pltpu
from jax.experimental.pallas import tpu_sc as plsc

VOCAB = 50257
HID = 2048
B = 4
S = 2048
NTOK = B * S

_info = plsc.get_sparse_core_info()
NC = _info.num_cores
NS = _info.num_subcores
NW = NC * NS
TPW = NTOK // NW
C = 8
G = TPW // C
VPR = HID // 16
NBUF = 2


def _make_kernel():
    mesh = plsc.VectorSubcoreMesh(core_axis_name="c", subcore_axis_name="s")

    @functools.partial(
        pl.kernel,
        mesh=mesh,
        out_type=jax.ShapeDtypeStruct((S, B * HID), jnp.float32),
        scratch_types=[
            pltpu.VMEM((G, C), jnp.int32),
            pltpu.VMEM((G, C), jnp.int32),
        ] + [pltpu.VMEM((C, HID), jnp.float32)] * (2 * NBUF)
          + [pltpu.VMEM((C // B, B * HID), jnp.float32)] * NBUF
          + [pltpu.SemaphoreType.DMA] * (3 * NBUF),
    )
    def emb_kernel(wids_hbm, pids_hbm, wtab_hbm, ptab_hbm, out_hbm,
                   widx_v, pidx_v,
                   wbuf0, wbuf1, pbuf0, pbuf1, obuf0, obuf1,
                   wsem0, wsem1, psem0, psem1, osem0, osem1):
        wbufs = (wbuf0, wbuf1)
        pbufs = (pbuf0, pbuf1)
        obufs = (obuf0, obuf1)
        wsems = (wsem0, wsem1)
        psems = (psem0, psem1)
        osems = (osem0, osem1)

        wid = lax.axis_index("s") * NC + lax.axis_index("c")
        sbase = wid * (TPW // B)
        pltpu.sync_copy(wids_hbm.at[wid], widx_v)
        pltpu.sync_copy(pids_hbm.at[wid], pidx_v)

        def start_gather(g, b):
            pltpu.async_copy(wtab_hbm.at[widx_v.at[g]], wbufs[b], wsems[b])
            pltpu.async_copy(ptab_hbm.at[pidx_v.at[g]], pbufs[b], psems[b])

        def wait_gather(b):
            pltpu.make_async_copy(
                wtab_hbm.at[pl.ds(0, C)], wbufs[b], wsems[b]).wait()
            pltpu.make_async_copy(
                ptab_hbm.at[pl.ds(0, C)], pbufs[b], psems[b]).wait()

        def wait_store(b):
            pltpu.make_async_copy(
                obufs[b], out_hbm.at[pl.ds(0, C // B)], osems[b]).wait()

        def do_add(b):
            def add_body(j, carry):
                col = j * 64
                for r in range(C):
                    for k in range(4):
                        cc = col + k * 16
                        obufs[b][r // B, pl.ds((r % B) * HID + cc, 16)] = (
                            wbufs[b][r, pl.ds(cc, 16)]
                            + pbufs[b][r, pl.ds(cc, 16)]
                        )
                return carry
            lax.fori_loop(0, VPR // 4, add_body, 0)

        def start_store(g, b):
            pltpu.async_copy(
                obufs[b],
                out_hbm.at[pl.ds(sbase + g * (C // B), C // B)],
                osems[b])

        for b in range(NBUF):
            start_gather(b, b)

        for b in range(NBUF):
            wait_gather(b)
            do_add(b)
            start_gather(NBUF + b, b)
            start_store(b, b)

        def step(i, carry):
            for b in range(NBUF):
                g = i * NBUF + b
                wait_gather(b)
                wait_store(b)
                do_add(b)
                start_gather(g + NBUF, b)
                start_store(g, b)
            return carry

        lax.fori_loop(1, G // NBUF - 1, step, 0)

        for b in range(NBUF):
            g = G - NBUF + b
            wait_gather(b)
            wait_store(b)
            do_add(b)
            start_store(g, b)

        for b in range(NBUF):
            wait_store(b)

    return emb_kernel


def kernel(input_ids, position_ids, word_embeddings, position_embeddings):
    ids = jnp.transpose(input_ids.astype(jnp.int32), (1, 0)).reshape(NW, G, C)
    pids = jnp.transpose(position_ids.astype(jnp.int32), (1, 0)).reshape(
        NW, G, C)
    out = _make_kernel()(ids, pids, word_embeddings, position_embeddings)
    return out.reshape(S, B, HID)


# speedup vs baseline: 2.0033x; 2.0033x over previous
import functools

import jax
import jax.numpy as jnp
from jax import lax
from jax.experimental import pallas as pl
from jax.experimental.pallas import tpu as pltpu
from jax.experimental.pallas import tpu_sc as plsc

VOCAB = 50257
HID = 2048
B = 4
S = 2048
NTOK = B * S

_info = plsc.get_sparse_core_info()
NC = _info.num_cores
NS = _info.num_subcores
NW = NC * NS
TPW = NTOK // NW
C = 8
G = TPW // C
VPR = HID // 16
NBUF = 2


def _make_kernel():
    mesh = plsc.VectorSubcoreMesh(core_axis_name="c", subcore_axis_name="s")

    @functools.partial(
        pl.kernel,
        mesh=mesh,
        out_type=jax.ShapeDtypeStruct((S, B, HID), jnp.float32),
        scratch_types=[
            pltpu.VMEM((G, C), jnp.int32),
            pltpu.VMEM((G, C), jnp.int32),
        ] + [pltpu.VMEM((C, HID), jnp.float32)] * (2 * NBUF)
          + [pltpu.VMEM((C, HID), jnp.float32)] * NBUF
          + [pltpu.SemaphoreType.DMA] * (3 * NBUF),
    )
    def emb_kernel(wids_hbm, pids_hbm, wtab_hbm, ptab_hbm, out_hbm,
                   widx_v, pidx_v,
                   wbuf0, wbuf1, pbuf0, pbuf1, obuf0, obuf1,
                   wsem0, wsem1, psem0, psem1, osem0, osem1):
        wbufs = (wbuf0, wbuf1)
        pbufs = (pbuf0, pbuf1)
        obufs = (obuf0, obuf1)
        wsems = (wsem0, wsem1)
        psems = (psem0, psem1)
        osems = (osem0, osem1)

        wid = lax.axis_index("s") * NC + lax.axis_index("c")
        sbase = wid * (TPW // B)
        pltpu.sync_copy(wids_hbm.at[wid], widx_v)
        pltpu.sync_copy(pids_hbm.at[wid], pidx_v)

        def start_gather(g, b):
            pltpu.async_copy(wtab_hbm.at[widx_v.at[g]], wbufs[b], wsems[b])
            pltpu.async_copy(ptab_hbm.at[pidx_v.at[g]], pbufs[b], psems[b])

        def wait_gather(b):
            pltpu.make_async_copy(
                wtab_hbm.at[pl.ds(0, C)], wbufs[b], wsems[b]).wait()
            pltpu.make_async_copy(
                ptab_hbm.at[pl.ds(0, C)], pbufs[b], psems[b]).wait()

        def wait_store(b):
            for h in range(C // B):
                pltpu.make_async_copy(
                    obufs[b].at[pl.ds(h * B, B)], out_hbm.at[0],
                    osems[b]).wait()

        def do_add(b):
            def add_body(j, carry):
                col = j * 64
                for r in range(C):
                    for k in range(4):
                        cc = col + k * 16
                        obufs[b][r, pl.ds(cc, 16)] = (
                            wbufs[b][r, pl.ds(cc, 16)]
                            + pbufs[b][r, pl.ds(cc, 16)]
                        )
                return carry
            lax.fori_loop(0, VPR // 4, add_body, 0)

        def start_store(g, b):
            for h in range(C // B):
                pltpu.async_copy(
                    obufs[b].at[pl.ds(h * B, B)],
                    out_hbm.at[sbase + g * (C // B) + h],
                    osems[b])

        for b in range(NBUF):
            start_gather(b, b)

        for b in range(NBUF):
            wait_gather(b)
            do_add(b)
            start_gather(NBUF + b, b)
            start_store(b, b)

        def step(i, carry):
            for b in range(NBUF):
                g = i * NBUF + b
                wait_gather(b)
                wait_store(b)
                do_add(b)
                start_gather(g + NBUF, b)
                start_store(g, b)
            return carry

        lax.fori_loop(1, G // NBUF - 1, step, 0)

        for b in range(NBUF):
            g = G - NBUF + b
            wait_gather(b)
            wait_store(b)
            do_add(b)
            start_store(g, b)

        for b in range(NBUF):
            wait_store(b)

    return emb_kernel


def kernel(input_ids, position_ids, word_embeddings, position_embeddings):
    ids = jnp.transpose(input_ids.astype(jnp.int32), (1, 0)).reshape(NW, G, C)
    pids = jnp.transpose(position_ids.astype(jnp.int32), (1, 0)).reshape(
        NW, G, C)
    return _make_kernel()(ids, pids, word_embeddings, position_embeddings)
